# baseline (speedup 1.0000x reference)
import functools

import jax
import jax.numpy as jnp
from jax import lax
from jax.experimental import pallas as pl
from jax.experimental.pallas import tpu as pltpu
from jax.experimental.pallas import tpu_sc as plsc

_NC = 2
_NS = 16
_L = 16
_K = 128


def _sc_kernel_body(N, E, D, R, P, nseg,
                    x_hbm, src_hbm, dst_hbm, et_hbm, pool_hbm, rep_hbm,
                    s_out, c_out, xp_out,
                    slotmap, src_s, dst_s, typ_s, packq,
                    rows, ones_rows, zbuf, pool_v, rep_v,
                    srcchunk, segchunk, s_acc, c_acc, sem):
    c = lax.axis_index("c")
    s = lax.axis_index("s")
    wid = c * _NS + s
    e_sh = E // (_NC * _NS)
    n_vec = e_sh // _L
    base = wid * e_sh

    pltpu.sync_copy(src_hbm.at[pl.ds(base, e_sh)], src_s)
    pltpu.sync_copy(dst_hbm.at[pl.ds(base, e_sh)], dst_s)
    pltpu.sync_copy(et_hbm.at[pl.ds(base, e_sh)], typ_s)
    pltpu.sync_copy(pool_hbm, pool_v)
    pltpu.sync_copy(rep_hbm, rep_v)

    one_hot = jnp.where(lax.iota(jnp.int32, _L) == 0,
                        jnp.float32(1), jnp.float32(0))
    zero16 = jnp.zeros((_L,), jnp.float32)

    zrows_half = nseg // _NS // 2 + 1

    def zrow_body(i, _):
        for k in range(D // _L):
            zbuf[i, pl.ds(k * _L, _L)] = zero16
        return 0

    lax.fori_loop(0, zrows_half, zrow_body, 0)

    def ones_body(i, _):
        ones_rows[i, pl.ds(0, _L)] = one_hot
        for k in range(1, D // _L):
            ones_rows[i, pl.ds(k * _L, _L)] = zero16
        return 0

    lax.fori_loop(0, _K, ones_body, 0)

    rows_per_tile = nseg // _NS
    zlo = s * rows_per_tile
    zrest = rows_per_tile - zrows_half
    pltpu.sync_copy(zbuf, s_acc.at[pl.ds(zlo, zrows_half)])
    pltpu.sync_copy(zbuf.at[pl.ds(0, zrest)],
                    s_acc.at[pl.ds(zlo + zrows_half, zrest)])
    pltpu.sync_copy(zbuf, c_acc.at[pl.ds(zlo, zrows_half)])
    pltpu.sync_copy(zbuf.at[pl.ds(0, zrest)],
                    c_acc.at[pl.ds(zlo + zrows_half, zrest)])

    def init_body(i, _):
        for k in range(5):
            slotmap[pl.ds((i * 5 + k) * _L, _L)] = jnp.full((_L,), -1,
                                                            jnp.int32)
        return 0

    lax.fori_loop(0, N // (_L * 5), init_body, 0)
    for j in range(P // _L):
        pv = pool_v[pl.ds(j * _L, _L)]
        rv = rep_v[pl.ds(j * _L, _L)]
        plsc.store_scatter(slotmap, [pv], rv)
    plsc.subcore_barrier()

    def scan_body(i, qptr):
        dv = dst_s[pl.ds(i * _L, _L)]
        sl = plsc.load_gather(slotmap, [dv])
        m = sl >= 0
        tv = typ_s[pl.ds(i * _L, _L)]
        sv = src_s[pl.ds(i * _L, _L)]
        seg = tv * P + sl
        pk = jnp.bitwise_or(sv, jnp.left_shift(seg, 14))
        plsc.store_compressed(packq.at[pl.ds(qptr, _L)], pk, mask=m)
        return qptr + jnp.sum(m.astype(jnp.int32))

    nq = lax.fori_loop(0, n_vec, scan_body, jnp.int32(0))

    all_on = lax.iota(jnp.int32, _L) >= 0
    padv = jnp.full((_L,), (R * P) << 14, jnp.int32)

    def pad_body(i, _):
        plsc.store_compressed(packq.at[pl.ds(nq + i * _L, _L)], padv,
                              mask=all_on)
        return 0

    lax.fori_loop(0, _K // _L, pad_body, 0)

    def chunk_body(j, _):
        for t in range(_K // _L):
            pk = packq[pl.ds(j * _K + t * _L, _L)]
            srcchunk[pl.ds(t * _L, _L)] = jnp.bitwise_and(pk, 16383)
            segchunk[pl.ds(t * _L, _L)] = lax.shift_right_logical(pk, 14)
        pltpu.async_copy(x_hbm.at[srcchunk], rows, sem).wait()
        pltpu.sync_copy(rows, s_acc.at[segchunk], add=True)
        pltpu.sync_copy(ones_rows, c_acc.at[segchunk], add=True)
        return 0

    nchunks = (nq + _K - 1) // _K
    lax.fori_loop(0, nchunks, chunk_body, 0)
    plsc.subcore_barrier()

    out_rows = (R * P) // _NS
    pltpu.sync_copy(s_acc.at[pl.ds(s * out_rows, out_rows)],
                    s_out.at[c, pl.ds(s * out_rows, out_rows)])
    pltpu.sync_copy(c_acc.at[pl.ds(s * out_rows, out_rows)],
                    c_out.at[c, pl.ds(s * out_rows, out_rows)])

    @pl.when(s == 0)
    def _():
        pltpu.async_copy(x_hbm.at[pool_v.at[pl.ds(c * _K, _K)]], rows,
                         sem).wait()
        pltpu.sync_copy(rows, xp_out.at[pl.ds(c * _K, _K)])


def _tc_body(R, P, invP, s_ref, c_ref, xp_ref, wr_ref, ws_ref, mult_ref,
             out_ref):
    seg_sum = s_ref[0] + s_ref[1]
    cnt = c_ref[0] + c_ref[1]
    norm = 1.0 / jnp.maximum(cnt[:, 0:1], 1.0)
    seg_n = seg_sum * norm
    agg = jnp.zeros_like(xp_ref)
    for r in range(R):
        agg = agg + jnp.dot(seg_n[r * P:(r + 1) * P, :], wr_ref[r],
                            preferred_element_type=jnp.float32)
    h = jnp.dot(xp_ref[...], ws_ref[...],
                preferred_element_type=jnp.float32) + agg
    relu = jnp.maximum(h, 0.0)
    out_ref[...] = jnp.dot(mult_ref[...], relu,
                           preferred_element_type=jnp.float32) * invP


def kernel(x, edge_index, edge_type, pool_indices, W_self, W_rel):
    N, D = x.shape
    E = edge_type.shape[0]
    R = W_rel.shape[0]
    P = pool_indices.shape[0]
    nseg_pad = R * P + _L
    nseg_pad += (-nseg_pad) % _NS
    e_sh = E // (_NC * _NS)

    eq = pool_indices[:, None] == pool_indices[None, :]
    rep = jnp.argmax(eq, axis=1).astype(jnp.int32)
    mult = jnp.sum(eq, axis=1)
    mult_eff = jnp.where(rep == jnp.arange(P, dtype=jnp.int32), mult,
                         0).astype(jnp.float32).reshape(1, P)

    mesh = plsc.VectorSubcoreMesh(core_axis_name="c", subcore_axis_name="s",
                                  num_cores=_NC, num_subcores=_NS)
    sc = pl.kernel(
        functools.partial(_sc_kernel_body, N, E, D, R, P, nseg_pad),
        out_type=[
            jax.ShapeDtypeStruct((_NC, R * P, D), jnp.float32),
            jax.ShapeDtypeStruct((_NC, R * P, D), jnp.float32),
            jax.ShapeDtypeStruct((P, D), jnp.float32),
        ],
        mesh=mesh,
        compiler_params=pltpu.CompilerParams(needs_layout_passes=False),
        scratch_types=[
            pltpu.VMEM((N,), jnp.int32),
            pltpu.VMEM((e_sh,), jnp.int32),
            pltpu.VMEM((e_sh,), jnp.int32),
            pltpu.VMEM((e_sh,), jnp.int32),
            pltpu.VMEM((e_sh + _K,), jnp.int32),
            pltpu.VMEM((_K, D), jnp.float32),
            pltpu.VMEM((_K, D), jnp.float32),
            pltpu.VMEM((nseg_pad // _NS // 2 + 1, D), jnp.float32),
            pltpu.VMEM((P,), jnp.int32),
            pltpu.VMEM((P,), jnp.int32),
            pltpu.VMEM((_K,), jnp.int32),
            pltpu.VMEM((_K,), jnp.int32),
            pltpu.VMEM_SHARED((nseg_pad, D), jnp.float32),
            pltpu.VMEM_SHARED((nseg_pad, D), jnp.float32),
            pltpu.SemaphoreType.DMA,
        ],
    )
    s_out, c_out, xp = sc(x, edge_index[0], edge_index[1], edge_type,
                          pool_indices, rep)

    tc = pl.pallas_call(
        functools.partial(_tc_body, R, P, 1.0 / P),
        out_shape=jax.ShapeDtypeStruct((1, D), jnp.float32),
    )
    return tc(s_out, c_out, xp, W_rel, W_self, mult_eff)


# speedup vs baseline: 1.7360x; 1.7360x over previous
import functools

import jax
import jax.numpy as jnp
from jax import lax
from jax.experimental import pallas as pl
from jax.experimental.pallas import tpu as pltpu
from jax.experimental.pallas import tpu_sc as plsc

_NC = 2
_NS = 16
_L = 16
_K = 128
_KQ = 16


def _sc_kernel_body(N, E, D, R, P, nseg,
                    x_hbm, src_hbm, dst_hbm, et_hbm, pool_hbm, rep_hbm,
                    s_out, c_out, xp_out,
                    slotmap, src_s, dst_s, typ_s, packq,
                    rows, ones_rows, zbuf, pool_v, rep_v,
                    srcchunk, segchunk, s_acc, c_acc, sem):
    c = lax.axis_index("c")
    s = lax.axis_index("s")
    wid = c * _NS + s
    e_sh = E // (_NC * _NS)
    n_vec = e_sh // _L
    base = wid * e_sh

    pltpu.sync_copy(src_hbm.at[pl.ds(base, e_sh)], src_s)
    pltpu.sync_copy(dst_hbm.at[pl.ds(base, e_sh)], dst_s)
    pltpu.sync_copy(et_hbm.at[pl.ds(base, e_sh)], typ_s)
    pltpu.sync_copy(pool_hbm, pool_v)
    pltpu.sync_copy(rep_hbm, rep_v)

    one_hot = jnp.where(lax.iota(jnp.int32, _L) == 0,
                        jnp.float32(1), jnp.float32(0))
    zero16 = jnp.zeros((_L,), jnp.float32)

    zrows_half = nseg // _NS // 2 + 1

    def zrow_body(i, _):
        for k in range(D // _L):
            zbuf[i, pl.ds(k * _L, _L)] = zero16
        return 0

    lax.fori_loop(0, zrows_half, zrow_body, 0)

    def ones_body(i, _):
        ones_rows[i, pl.ds(0, _L)] = one_hot
        for k in range(1, D // _L):
            ones_rows[i, pl.ds(k * _L, _L)] = zero16
        return 0

    lax.fori_loop(0, _KQ, ones_body, 0)

    rows_per_tile = nseg // _NS
    zlo = s * rows_per_tile
    zrest = rows_per_tile - zrows_half
    pltpu.sync_copy(zbuf, s_acc.at[pl.ds(zlo, zrows_half)])
    pltpu.sync_copy(zbuf.at[pl.ds(0, zrest)],
                    s_acc.at[pl.ds(zlo + zrows_half, zrest)])
    pltpu.sync_copy(zbuf, c_acc.at[pl.ds(zlo, zrows_half)])
    pltpu.sync_copy(zbuf.at[pl.ds(0, zrest)],
                    c_acc.at[pl.ds(zlo + zrows_half, zrest)])

    def init_body(i, _):
        for k in range(5):
            slotmap[pl.ds((i * 5 + k) * _L, _L)] = jnp.full((_L,), -1,
                                                            jnp.int32)
        return 0

    lax.fori_loop(0, N // (_L * 5), init_body, 0)
    for j in range(P // _L):
        pv = pool_v[pl.ds(j * _L, _L)]
        rv = rep_v[pl.ds(j * _L, _L)]
        plsc.store_scatter(slotmap, [pv], rv)
    plsc.subcore_barrier()

    def scan_body(i, qptr):
        dv = dst_s[pl.ds(i * _L, _L)]
        sl = plsc.load_gather(slotmap, [dv])
        m = sl >= 0
        tv = typ_s[pl.ds(i * _L, _L)]
        sv = src_s[pl.ds(i * _L, _L)]
        seg = tv * P + sl
        pk = jnp.bitwise_or(sv, jnp.left_shift(seg, 14))
        plsc.store_compressed(packq.at[pl.ds(qptr, _L)], pk, mask=m)
        return qptr + jnp.sum(m.astype(jnp.int32))

    nq = lax.fori_loop(0, n_vec, scan_body, jnp.int32(0))

    all_on = lax.iota(jnp.int32, _L) >= 0
    padv = jnp.full((_L,), (R * P) << 14, jnp.int32)

    def pad_body(i, _):
        plsc.store_compressed(packq.at[pl.ds(nq + i * _L, _L)], padv,
                              mask=all_on)
        return 0

    lax.fori_loop(0, _KQ // _L, pad_body, 0)

    def chunk_body(j, _):
        for t in range(_KQ // _L):
            pk = packq[pl.ds(j * _KQ + t * _L, _L)]
            srcchunk[pl.ds(t * _L, _L)] = jnp.bitwise_and(pk, 16383)
            segchunk[pl.ds(t * _L, _L)] = lax.shift_right_logical(pk, 14)
        pltpu.async_copy(x_hbm.at[srcchunk], rows.at[pl.ds(0, _KQ)],
                         sem).wait()
        pltpu.sync_copy(rows.at[pl.ds(0, _KQ)], s_acc.at[segchunk], add=True)
        pltpu.sync_copy(ones_rows, c_acc.at[segchunk], add=True)
        return 0

    nchunks = (nq + _KQ - 1) // _KQ
    lax.fori_loop(0, nchunks, chunk_body, 0)
    plsc.subcore_barrier()

    out_rows = (R * P) // _NS
    pltpu.sync_copy(s_acc.at[pl.ds(s * out_rows, out_rows)],
                    s_out.at[c, pl.ds(s * out_rows, out_rows)])
    pltpu.sync_copy(c_acc.at[pl.ds(s * out_rows, out_rows)],
                    c_out.at[c, pl.ds(s * out_rows, out_rows)])

    @pl.when(s == 0)
    def _():
        pltpu.async_copy(x_hbm.at[pool_v.at[pl.ds(c * _K, _K)]], rows,
                         sem).wait()
        pltpu.sync_copy(rows, xp_out.at[pl.ds(c * _K, _K)])


def _tc_body(R, P, invP, s_ref, c_ref, xp_ref, wr_ref, ws_ref, mult_ref,
             out_ref):
    seg_sum = s_ref[0] + s_ref[1]
    cnt = c_ref[0] + c_ref[1]
    norm = 1.0 / jnp.maximum(cnt[:, 0:1], 1.0)
    seg_n = seg_sum * norm
    agg = jnp.zeros_like(xp_ref)
    for r in range(R):
        agg = agg + jnp.dot(seg_n[r * P:(r + 1) * P, :], wr_ref[r],
                            preferred_element_type=jnp.float32)
    h = jnp.dot(xp_ref[...], ws_ref[...],
                preferred_element_type=jnp.float32) + agg
    relu = jnp.maximum(h, 0.0)
    out_ref[...] = jnp.dot(mult_ref[...], relu,
                           preferred_element_type=jnp.float32) * invP


def kernel(x, edge_index, edge_type, pool_indices, W_self, W_rel):
    N, D = x.shape
    E = edge_type.shape[0]
    R = W_rel.shape[0]
    P = pool_indices.shape[0]
    nseg_pad = R * P + _L
    nseg_pad += (-nseg_pad) % _NS
    e_sh = E // (_NC * _NS)

    eq = pool_indices[:, None] == pool_indices[None, :]
    rep = jnp.argmax(eq, axis=1).astype(jnp.int32)
    mult = jnp.sum(eq, axis=1)
    mult_eff = jnp.where(rep == jnp.arange(P, dtype=jnp.int32), mult,
                         0).astype(jnp.float32).reshape(1, P)

    mesh = plsc.VectorSubcoreMesh(core_axis_name="c", subcore_axis_name="s",
                                  num_cores=_NC, num_subcores=_NS)
    sc = pl.kernel(
        functools.partial(_sc_kernel_body, N, E, D, R, P, nseg_pad),
        out_type=[
            jax.ShapeDtypeStruct((_NC, R * P, D), jnp.float32),
            jax.ShapeDtypeStruct((_NC, R * P, D), jnp.float32),
            jax.ShapeDtypeStruct((P, D), jnp.float32),
        ],
        mesh=mesh,
        compiler_params=pltpu.CompilerParams(needs_layout_passes=False),
        scratch_types=[
            pltpu.VMEM((N,), jnp.int32),
            pltpu.VMEM((e_sh,), jnp.int32),
            pltpu.VMEM((e_sh,), jnp.int32),
            pltpu.VMEM((e_sh,), jnp.int32),
            pltpu.VMEM((e_sh + _KQ,), jnp.int32),
            pltpu.VMEM((_K, D), jnp.float32),
            pltpu.VMEM((_KQ, D), jnp.float32),
            pltpu.VMEM((nseg_pad // _NS // 2 + 1, D), jnp.float32),
            pltpu.VMEM((P,), jnp.int32),
            pltpu.VMEM((P,), jnp.int32),
            pltpu.VMEM((_KQ,), jnp.int32),
            pltpu.VMEM((_KQ,), jnp.int32),
            pltpu.VMEM_SHARED((nseg_pad, D), jnp.float32),
            pltpu.VMEM_SHARED((nseg_pad, D), jnp.float32),
            pltpu.SemaphoreType.DMA,
        ],
    )
    s_out, c_out, xp = sc(x, edge_index[0], edge_index[1], edge_type,
                          pool_indices, rep)

    tc = pl.pallas_call(
        functools.partial(_tc_body, R, P, 1.0 / P),
        out_shape=jax.ShapeDtypeStruct((1, D), jnp.float32),
    )
    return tc(s_out, c_out, xp, W_rel, W_self, mult_eff)
